# baseline (device time: 143145 ns/iter reference)
import jax
import jax.numpy as jnp
from jax import lax
from jax.experimental import pallas as pl
from jax.experimental.pallas import tpu as pltpu

N_DEV = 32
SQ = 1024
SKV = 1024
DH = 128
HL = 8
DMODEL = 1024
CH = SQ // N_DEV
NBLK = 4
BR = SQ // NBLK
SCALE = 0.08838834764831843
NEG = -1e9


def kernel(x, Wq, K_ext, V_ext, Wo):
    x2 = x.reshape(SQ, DMODEL)

    def body(x_ref, wq_ref, k_hbm, v_hbm, wo_ref, out_ref,
             kbuf, vbuf, ctx_ref, bias_ref, part_ref,
             kv_sems, rs_buf, rs_send_sems, rs_recv_sems,
             ag_send_sems, ag_recv_sems):
        my = lax.axis_index("i")
        g4 = my // 8

        barrier_sem = pltpu.get_barrier_semaphore()
        for j in range(1, N_DEV):
            tgt = jax.lax.rem(my + j, N_DEV)
            pl.semaphore_signal(barrier_sem, inc=1, device_id=(tgt,),
                                device_id_type=pl.DeviceIdType.MESH)
        pl.semaphore_wait(barrier_sem, N_DEV - 1)

        kv_copies = []
        for h in range(HL):
            gh = my * HL + h
            ck = pltpu.make_async_copy(
                k_hbm.at[0, :, gh, :], kbuf.at[h], kv_sems.at[0, h])
            cv = pltpu.make_async_copy(
                v_hbm.at[0, :, gh, :], vbuf.at[h], kv_sems.at[1, h])
            ck.start()
            cv.start()
            kv_copies.append((ck, cv))

        qi0 = lax.broadcasted_iota(jnp.int32, (BR, SKV), 0)
        ki0 = lax.broadcasted_iota(jnp.int32, (BR, SKV), 1)
        mask0 = (jnp.abs(qi0 - ki0) <= 128) | (ki0 < 32) | (qi0 < 32)
        bias_ref[...] = jnp.where(mask0, 0.0, NEG).astype(jnp.float32)

        for h in range(HL):
            ck, cv = kv_copies[h]
            ck.wait()
            cv.wait()

        def send_block(b, rs_sends):
            for jj in range(8):
                t = b * 8 + jj
                rdma = pltpu.make_async_remote_copy(
                    src_ref=part_ref.at[pl.ds(t * CH, CH), :],
                    dst_ref=rs_buf.at[my],
                    send_sem=rs_send_sems.at[t],
                    recv_sem=rs_recv_sems.at[my],
                    device_id=(t,),
                    device_id_type=pl.DeviceIdType.MESH,
                )

                @pl.when(t != my)
                def _(rdma=rdma):
                    rdma.start()

                rs_sends.append((t, rdma))

        rs_sends = []

        qb = jnp.dot(x_ref[pl.ds(0, BR), :], wq_ref[...],
                     preferred_element_type=jnp.float32)
        for h in range(HL):
            scores = lax.dot_general(
                qb[:, h * DH:(h + 1) * DH], kbuf[h],
                (((1,), (1,)), ((), ())),
                preferred_element_type=jnp.float32) * SCALE
            e = jnp.exp(scores + bias_ref[...])
            w = e / jnp.sum(e, axis=1, keepdims=True)
            ctx_ref[:, h * DH:(h + 1) * DH] = jnp.dot(
                w, vbuf[h], preferred_element_type=jnp.float32)
        part_ref[pl.ds(0, BR), :] = jnp.dot(
            ctx_ref[...], wo_ref[...], preferred_element_type=jnp.float32)
        send_block(0, rs_sends)

        WW = 512
        for k in range(3):
            b = 1 + jax.lax.rem(g4 + k, 3)
            r0 = b * BR
            ws = jnp.minimum(r0 - 128, SKV - WW)
            d = r0 - ws
            iw = lax.broadcasted_iota(jnp.int32, (BR, WW), 0)
            jw = lax.broadcasted_iota(jnp.int32, (BR, WW), 1)
            wbias = jnp.where(jnp.abs(iw - jw + d) <= 128, 0.0, NEG)
            wbias = wbias.astype(jnp.float32)
            qb = jnp.dot(x_ref[pl.ds(r0, BR), :], wq_ref[...],
                         preferred_element_type=jnp.float32)
            for h in range(HL):
                qh = qb[:, h * DH:(h + 1) * DH]
                kw = kbuf[h, pl.ds(ws, WW), :]
                s_w = lax.dot_general(
                    qh, kw, (((1,), (1,)), ((), ())),
                    preferred_element_type=jnp.float32) * SCALE
                s_g = lax.dot_general(
                    qh, kbuf[h, 0:32, :], (((1,), (1,)), ((), ())),
                    preferred_element_type=jnp.float32) * SCALE
                e_w = jnp.exp(s_w + wbias)
                e_g = jnp.exp(s_g)
                denom = (jnp.sum(e_w, axis=1, keepdims=True)
                         + jnp.sum(e_g, axis=1, keepdims=True))
                ctx_ref[:, h * DH:(h + 1) * DH] = (
                    jnp.dot(e_w / denom, vbuf[h, pl.ds(ws, WW), :],
                            preferred_element_type=jnp.float32)
                    + jnp.dot(e_g / denom, vbuf[h, 0:32, :],
                              preferred_element_type=jnp.float32))
            part_ref[pl.ds(r0, BR), :] = jnp.dot(
                ctx_ref[...], wo_ref[...],
                preferred_element_type=jnp.float32)
            send_block(b, rs_sends)

        for i in range(1, N_DEV):
            src = jax.lax.rem(my + i, N_DEV)
            recv = pltpu.make_async_remote_copy(
                src_ref=rs_buf.at[src],
                dst_ref=rs_buf.at[src],
                send_sem=rs_send_sems.at[0],
                recv_sem=rs_recv_sems.at[src],
                device_id=(my,),
                device_id_type=pl.DeviceIdType.MESH,
            )
            recv.wait_recv()
        vals = [part_ref[pl.ds(my * CH, CH), :]]
        for i in range(1, N_DEV):
            src = jax.lax.rem(my + i, N_DEV)
            vals.append(rs_buf[src])
        while len(vals) > 1:
            nxt = [vals[i] + vals[i + 1] for i in range(0, len(vals) - 1, 2)]
            if len(vals) % 2:
                nxt.append(vals[-1])
            vals = nxt
        acc = vals[0]
        out_ref[pl.ds(my * CH, CH), :] = acc
        for t, rdma in rs_sends:
            @pl.when(t != my)
            def _(rdma=rdma):
                rdma.wait_send()

        ag_sends = []
        for j in range(1, N_DEV):
            t = jax.lax.rem(my + j, N_DEV)
            rdma = pltpu.make_async_remote_copy(
                src_ref=out_ref.at[pl.ds(my * CH, CH), :],
                dst_ref=out_ref.at[pl.ds(my * CH, CH), :],
                send_sem=ag_send_sems.at[j],
                recv_sem=ag_recv_sems.at[my],
                device_id=(t,),
                device_id_type=pl.DeviceIdType.MESH,
            )
            rdma.start()
            ag_sends.append(rdma)
        for i in range(1, N_DEV):
            src = jax.lax.rem(my + i, N_DEV)
            recv = pltpu.make_async_remote_copy(
                src_ref=out_ref.at[pl.ds(src * CH, CH), :],
                dst_ref=out_ref.at[pl.ds(src * CH, CH), :],
                send_sem=ag_send_sems.at[0],
                recv_sem=ag_recv_sems.at[src],
                device_id=(my,),
                device_id_type=pl.DeviceIdType.MESH,
            )
            recv.wait_recv()
        for rdma in ag_sends:
            rdma.wait_send()

    out = pl.pallas_call(
        body,
        out_shape=jax.ShapeDtypeStruct((SQ, DMODEL), jnp.float32),
        in_specs=[
            pl.BlockSpec(memory_space=pltpu.VMEM),
            pl.BlockSpec(memory_space=pltpu.VMEM),
            pl.BlockSpec(memory_space=pl.ANY),
            pl.BlockSpec(memory_space=pl.ANY),
            pl.BlockSpec(memory_space=pltpu.VMEM),
        ],
        out_specs=pl.BlockSpec(memory_space=pltpu.VMEM),
        scratch_shapes=[
            pltpu.VMEM((HL, SKV, DH), jnp.float32),
            pltpu.VMEM((HL, SKV, DH), jnp.float32),
            pltpu.VMEM((BR, HL * DH), jnp.float32),
            pltpu.VMEM((BR, SKV), jnp.float32),
            pltpu.VMEM((SQ, DMODEL), jnp.float32),
            pltpu.SemaphoreType.DMA((2, HL)),
            pltpu.VMEM((N_DEV, CH, DMODEL), jnp.float32),
            pltpu.SemaphoreType.DMA((N_DEV,)),
            pltpu.SemaphoreType.DMA((N_DEV,)),
            pltpu.SemaphoreType.DMA((N_DEV,)),
            pltpu.SemaphoreType.DMA((N_DEV,)),
        ],
        compiler_params=pltpu.CompilerParams(collective_id=0),
    )(x2, Wq, K_ext, V_ext, Wo)
    return out.reshape(1, SQ, DMODEL)


# device time: 130801 ns/iter; 1.0944x vs baseline; 1.0944x over previous
import jax
import jax.numpy as jnp
from jax import lax
from jax.experimental import pallas as pl
from jax.experimental.pallas import tpu as pltpu

N_DEV = 32
SQ = 1024
SKV = 1024
DH = 128
HL = 8
DMODEL = 1024
CH = SQ // N_DEV
NBLK = 4
BR = SQ // NBLK
SCALE = 0.08838834764831843
NEG = -1e9


def kernel(x, Wq, K_ext, V_ext, Wo):
    x2 = x.reshape(SQ, DMODEL)

    def body(x_ref, wq_ref, k_hbm, v_hbm, wo_ref, out_ref,
             kbuf, vbuf, ctx_ref, bias_ref, part_ref,
             kv_sems, rs_buf, rs_send_sems, rs_recv_sems,
             ag_send_sems, ag_recv_sems):
        my = lax.axis_index("i")
        g4 = my // 8

        barrier_sem = pltpu.get_barrier_semaphore()
        for j in range(1, N_DEV):
            tgt = jax.lax.rem(my + j, N_DEV)
            pl.semaphore_signal(barrier_sem, inc=1, device_id=(tgt,),
                                device_id_type=pl.DeviceIdType.MESH)
        pl.semaphore_wait(barrier_sem, N_DEV - 1)

        kv_copies = []
        for h in range(HL):
            gh = my * HL + h
            ck = pltpu.make_async_copy(
                k_hbm.at[0, :, gh, :], kbuf.at[h], kv_sems.at[0, h])
            cv = pltpu.make_async_copy(
                v_hbm.at[0, :, gh, :], vbuf.at[h], kv_sems.at[1, h])
            ck.start()
            cv.start()
            kv_copies.append((ck, cv))

        qi = lax.broadcasted_iota(jnp.int32, (SQ, SKV), 0)
        ki = lax.broadcasted_iota(jnp.int32, (SQ, SKV), 1)
        mask = (jnp.abs(qi - ki) <= 128) | (ki < 32) | (qi < 32)
        bias_ref[...] = jnp.where(mask, 0.0, NEG).astype(jnp.float32)

        for h in range(HL):
            ck, cv = kv_copies[h]
            ck.wait()
            cv.wait()

        rs_sends = []
        for k in range(NBLK):
            b = jax.lax.rem(g4 + 1 + k, NBLK)
            r0 = b * BR
            qb = jnp.dot(x_ref[pl.ds(r0, BR), :], wq_ref[...],
                         preferred_element_type=jnp.float32)
            for h in range(HL):
                scores = lax.dot_general(
                    qb[:, h * DH:(h + 1) * DH], kbuf[h],
                    (((1,), (1,)), ((), ())),
                    preferred_element_type=jnp.float32) * SCALE
                scores = scores + bias_ref[pl.ds(r0, BR), :]
                e = jnp.exp(scores)
                w = e / jnp.sum(e, axis=1, keepdims=True)
                ctx_ref[:, h * DH:(h + 1) * DH] = jnp.dot(
                    w, vbuf[h], preferred_element_type=jnp.float32)
            part_ref[pl.ds(r0, BR), :] = jnp.dot(
                ctx_ref[...], wo_ref[...],
                preferred_element_type=jnp.float32)
            for jj in range(8):
                t = b * 8 + jj
                rdma = pltpu.make_async_remote_copy(
                    src_ref=part_ref.at[pl.ds(t * CH, CH), :],
                    dst_ref=rs_buf.at[my],
                    send_sem=rs_send_sems.at[t],
                    recv_sem=rs_recv_sems.at[my],
                    device_id=(t,),
                    device_id_type=pl.DeviceIdType.MESH,
                )

                @pl.when(t != my)
                def _(rdma=rdma):
                    rdma.start()

                rs_sends.append((t, rdma))

        for i in range(1, N_DEV):
            src = jax.lax.rem(my + i, N_DEV)
            recv = pltpu.make_async_remote_copy(
                src_ref=rs_buf.at[src],
                dst_ref=rs_buf.at[src],
                send_sem=rs_send_sems.at[0],
                recv_sem=rs_recv_sems.at[src],
                device_id=(my,),
                device_id_type=pl.DeviceIdType.MESH,
            )
            recv.wait_recv()
        vals = [part_ref[pl.ds(my * CH, CH), :]]
        for i in range(1, N_DEV):
            src = jax.lax.rem(my + i, N_DEV)
            vals.append(rs_buf[src])
        while len(vals) > 1:
            nxt = [vals[i] + vals[i + 1] for i in range(0, len(vals) - 1, 2)]
            if len(vals) % 2:
                nxt.append(vals[-1])
            vals = nxt
        out_ref[pl.ds(my * CH, CH), :] = vals[0]
        for t, rdma in rs_sends:
            @pl.when(t != my)
            def _(rdma=rdma):
                rdma.wait_send()

        ag_sends = []
        for j in range(1, N_DEV):
            t = jax.lax.rem(my + j, N_DEV)
            rdma = pltpu.make_async_remote_copy(
                src_ref=out_ref.at[pl.ds(my * CH, CH), :],
                dst_ref=out_ref.at[pl.ds(my * CH, CH), :],
                send_sem=ag_send_sems.at[j],
                recv_sem=ag_recv_sems.at[my],
                device_id=(t,),
                device_id_type=pl.DeviceIdType.MESH,
            )
            rdma.start()
            ag_sends.append(rdma)
        for i in range(1, N_DEV):
            src = jax.lax.rem(my + i, N_DEV)
            recv = pltpu.make_async_remote_copy(
                src_ref=out_ref.at[pl.ds(src * CH, CH), :],
                dst_ref=out_ref.at[pl.ds(src * CH, CH), :],
                send_sem=ag_send_sems.at[0],
                recv_sem=ag_recv_sems.at[src],
                device_id=(my,),
                device_id_type=pl.DeviceIdType.MESH,
            )
            recv.wait_recv()
        for rdma in ag_sends:
            rdma.wait_send()

    out = pl.pallas_call(
        body,
        out_shape=jax.ShapeDtypeStruct((SQ, DMODEL), jnp.float32),
        in_specs=[
            pl.BlockSpec(memory_space=pltpu.VMEM),
            pl.BlockSpec(memory_space=pltpu.VMEM),
            pl.BlockSpec(memory_space=pl.ANY),
            pl.BlockSpec(memory_space=pl.ANY),
            pl.BlockSpec(memory_space=pltpu.VMEM),
        ],
        out_specs=pl.BlockSpec(memory_space=pltpu.VMEM),
        scratch_shapes=[
            pltpu.VMEM((HL, SKV, DH), jnp.float32),
            pltpu.VMEM((HL, SKV, DH), jnp.float32),
            pltpu.VMEM((BR, HL * DH), jnp.float32),
            pltpu.VMEM((SQ, SKV), jnp.float32),
            pltpu.VMEM((SQ, DMODEL), jnp.float32),
            pltpu.SemaphoreType.DMA((2, HL)),
            pltpu.VMEM((N_DEV, CH, DMODEL), jnp.float32),
            pltpu.SemaphoreType.DMA((N_DEV,)),
            pltpu.SemaphoreType.DMA((N_DEV,)),
            pltpu.SemaphoreType.DMA((N_DEV,)),
            pltpu.SemaphoreType.DMA((N_DEV,)),
        ],
        compiler_params=pltpu.CompilerParams(collective_id=0),
    )(x2, Wq, K_ext, V_ext, Wo)
    return out.reshape(1, SQ, DMODEL)
